# baseline (speedup 1.0000x reference)
import functools
import math

import jax
import jax.numpy as jnp
from jax import lax
from jax.experimental import pallas as pl
from jax.experimental.pallas import tpu as pltpu
from jax.experimental.pallas import tpu_sc as plsc

DIM = 64
SEQ_LEN = 200
BATCH = 4096
NC = 2
NS = 16
NW = NC * NS
CHUNK = 128
SCALE = math.sqrt(DIM)


def _tc_widen_scale(table):
    n = table.shape[0]
    blk = 8000
    grid = n // blk

    def body(i_ref, o_ref):
        o_ref[:, pl.ds(0, DIM)] = i_ref[...] * SCALE

    return pl.pallas_call(
        body,
        grid=(grid,),
        in_specs=[pl.BlockSpec((blk, DIM), lambda i: (i, 0))],
        out_specs=pl.BlockSpec((blk, 128), lambda i: (i, 0)),
        out_shape=jax.ShapeDtypeStruct((n, 128), jnp.float32),
    )(table)


def _make_sc_gather(n_flat):
    per_w = n_flat // NW
    n_chunks = per_w // CHUNK
    assert n_chunks % 2 == 0
    mesh = plsc.VectorSubcoreMesh(core_axis_name="c", subcore_axis_name="s")

    @functools.partial(
        pl.kernel,
        mesh=mesh,
        out_type=jax.ShapeDtypeStruct((n_flat // 512, 32, 8, 128), jnp.float32),
        compiler_params=pltpu.CompilerParams(
            use_tc_tiling_on_sc=False, needs_layout_passes=False),
        scratch_types=[
            pltpu.VMEM((n_chunks, CHUNK), jnp.int32),
            pltpu.VMEM((CHUNK, 128), jnp.float32),
            pltpu.VMEM((CHUNK, 128), jnp.float32),
            pltpu.VMEM((8, 8, 133), jnp.float32),
            pltpu.VMEM((8, 8, 133), jnp.float32),
            pltpu.VMEM((DIM,), jnp.float32),
            pltpu.SemaphoreType.DMA,
            pltpu.SemaphoreType.DMA,
            pltpu.SemaphoreType.DMA,
            pltpu.SemaphoreType.DMA,
        ],
    )
    def body(idx_hbm, table_hbm, pe_hbm, out_hbm, idx_v,
             slab0, slab1, outb0, outb1, peb, sg0, sg1, sw0, sw1):
        wid = lax.axis_index("c") * NS + lax.axis_index("s")
        w_row = wid * n_chunks
        w_base = wid * per_w
        slabs = (slab0, slab1)
        outbs = (outb0, outb1)
        sg = (sg0, sg1)
        sw = (sw0, sw1)
        iota = lax.iota(jnp.int32, 16)
        dks = [iota + 16 * k for k in range(4)]
        i0s = [d >> 3 for d in dks]
        i1s = [d & 7 for d in dks]

        pltpu.sync_copy(idx_hbm.at[pl.ds(w_row, n_chunks)], idx_v)

        def gather(c, b):
            pltpu.async_copy(table_hbm.at[idx_v.at[c]], slabs[b], sg[b])

        def wait_g(b):
            pltpu.make_async_copy(
                table_hbm.at[idx_v.at[0]], slabs[b], sg[b]).wait()

        def wait_w(b):
            pltpu.make_async_copy(
                outbs[b].at[:, :, pl.ds(0, 128)],
                out_hbm.at[pl.ds(0, 8), 0], sw[b]).wait()

        s0 = w_base // BATCH
        pltpu.sync_copy(pe_hbm.at[s0], peb)
        gather(0, 0)

        def outer(c2, s_prev):
            for b in (0, 1):
                c = c2 * 2 + b
                q = 1 - b
                base = w_base + c * CHUNK
                s = base // BATCH
                cb = (base // CHUNK) % 32

                @pl.when(c + 1 < n_chunks)
                def _():
                    gather(c + 1, q)

                @pl.when(s != s_prev)
                def _():
                    pltpu.sync_copy(pe_hbm.at[s], peb)

                wait_g(b)

                @pl.when(c >= 2)
                def _():
                    wait_w(b)

                slab = slabs[b]
                outb = outbs[b]
                pes = [peb[pl.ds(16 * k, 16)] for k in range(4)]

                @plsc.parallel_loop(0, CHUNK, 1, unroll=4)
                def _(r):
                    rv = jnp.broadcast_to(r, (16,)).astype(jnp.int32)
                    for k in range(4):
                        vals = slab[r, pl.ds(16 * k, 16)] + pes[k]
                        plsc.store_scatter(outb, [i0s[k], i1s[k], rv], vals)

                pltpu.async_copy(outb.at[:, :, pl.ds(0, 128)],
                                 out_hbm.at[pl.ds(s * 8, 8), cb], sw[b])
                s_prev = s
            return s_prev

        lax.fori_loop(0, n_chunks // 2, outer, s0)
        wait_w(0)
        wait_w(1)

    return body


def kernel(x, emb_weight, positional_encodings):
    seq, batch = x.shape
    idx2d = x.reshape(-1, CHUNK)
    pe2d = positional_encodings.reshape(positional_encodings.shape[0], DIM)[:seq]
    table_wide = _tc_widen_scale(emb_weight)
    out4 = _make_sc_gather(seq * batch)(idx2d, table_wide, pe2d)
    out = (out4.reshape(seq, 8, 32, 8, 128)
               .transpose(0, 2, 4, 1, 3)
               .reshape(seq, batch, DIM))
    return out


# speedup vs baseline: 1.8842x; 1.8842x over previous
import functools
import math

import jax
import jax.numpy as jnp
from jax import lax
from jax.experimental import pallas as pl
from jax.experimental.pallas import tpu as pltpu
from jax.experimental.pallas import tpu_sc as plsc

DIM = 64
SEQ_LEN = 200
BATCH = 4096
NC = 2
NS = 16
NW = NC * NS
CHUNK = 128
SCALE = math.sqrt(DIM)


def _tc_widen_scale(table):
    n = table.shape[0]
    blk = 4096
    grid = pl.cdiv(n, blk)

    def body(i_ref, o_ref):
        o_ref[:, pl.ds(0, DIM)] = i_ref[...].T * SCALE

    return pl.pallas_call(
        body,
        grid=(grid,),
        in_specs=[pl.BlockSpec((DIM, blk), lambda i: (0, i))],
        out_specs=pl.BlockSpec((blk, 128), lambda i: (i, 0)),
        out_shape=jax.ShapeDtypeStruct((n, 128), jnp.float32),
    )(table.T)


def _make_sc_gather(n_flat):
    per_w = n_flat // NW
    n_chunks = per_w // CHUNK
    assert n_chunks % 2 == 0
    mesh = plsc.VectorSubcoreMesh(core_axis_name="c", subcore_axis_name="s")

    @functools.partial(
        pl.kernel,
        mesh=mesh,
        out_type=jax.ShapeDtypeStruct((n_flat // 512, 32, 8, 128), jnp.float32),
        compiler_params=pltpu.CompilerParams(
            use_tc_tiling_on_sc=False, needs_layout_passes=False),
        scratch_types=[
            pltpu.VMEM((n_chunks, CHUNK), jnp.int32),
            pltpu.VMEM((CHUNK, 128), jnp.float32),
            pltpu.VMEM((CHUNK, 128), jnp.float32),
            pltpu.VMEM((8, 8, 133), jnp.float32),
            pltpu.VMEM((8, 8, 133), jnp.float32),
            pltpu.VMEM((DIM,), jnp.float32),
            pltpu.SemaphoreType.DMA,
            pltpu.SemaphoreType.DMA,
            pltpu.SemaphoreType.DMA,
            pltpu.SemaphoreType.DMA,
        ],
    )
    def body(idx_hbm, table_hbm, pe_hbm, out_hbm, idx_v,
             slab0, slab1, outb0, outb1, peb, sg0, sg1, sw0, sw1):
        wid = lax.axis_index("c") * NS + lax.axis_index("s")
        w_row = wid * n_chunks
        w_base = wid * per_w
        slabs = (slab0, slab1)
        outbs = (outb0, outb1)
        sg = (sg0, sg1)
        sw = (sw0, sw1)
        iota = lax.iota(jnp.int32, 16)
        dks = [iota + 16 * k for k in range(4)]
        i0s = [d >> 3 for d in dks]
        i1s = [d & 7 for d in dks]

        pltpu.sync_copy(idx_hbm.at[pl.ds(w_row, n_chunks)], idx_v)

        def gather(c, b):
            pltpu.async_copy(table_hbm.at[idx_v.at[c]], slabs[b], sg[b])

        def wait_g(b):
            pltpu.make_async_copy(
                table_hbm.at[idx_v.at[0]], slabs[b], sg[b]).wait()

        def wait_w(b):
            pltpu.make_async_copy(
                outbs[b].at[:, :, pl.ds(0, 128)],
                out_hbm.at[pl.ds(0, 8), 0], sw[b]).wait()

        s0 = w_base // BATCH
        pltpu.sync_copy(pe_hbm.at[s0], peb)
        gather(0, 0)

        def outer(c2, s_prev):
            for b in (0, 1):
                c = c2 * 2 + b
                q = 1 - b
                base = w_base + c * CHUNK
                s = base // BATCH
                cb = (base // CHUNK) % 32

                @pl.when(c + 1 < n_chunks)
                def _():
                    gather(c + 1, q)

                @pl.when(s != s_prev)
                def _():
                    pltpu.sync_copy(pe_hbm.at[s], peb)

                wait_g(b)

                @pl.when(c >= 2)
                def _():
                    wait_w(b)

                slab = slabs[b]
                outb = outbs[b]
                pes = [peb[pl.ds(16 * k, 16)] for k in range(4)]

                @plsc.parallel_loop(0, CHUNK, 1, unroll=4)
                def _(r):
                    rv = jnp.broadcast_to(r, (16,)).astype(jnp.int32)
                    for k in range(4):
                        vals = slab[r, pl.ds(16 * k, 16)] + pes[k]
                        plsc.store_scatter(outb, [i0s[k], i1s[k], rv], vals)

                pltpu.async_copy(outb.at[:, :, pl.ds(0, 128)],
                                 out_hbm.at[pl.ds(s * 8, 8), cb], sw[b])
                s_prev = s
            return s_prev

        lax.fori_loop(0, n_chunks // 2, outer, s0)
        wait_w(0)
        wait_w(1)

    return body


def kernel(x, emb_weight, positional_encodings):
    seq, batch = x.shape
    idx2d = x.reshape(-1, CHUNK)
    pe2d = positional_encodings.reshape(positional_encodings.shape[0], DIM)[:seq]
    table_wide = _tc_widen_scale(emb_weight)
    out4 = _make_sc_gather(seq * batch)(idx2d, table_wide, pe2d)
    out = (out4.reshape(seq, 8, 32, 8, 128)
               .transpose(0, 2, 4, 1, 3)
               .reshape(seq, batch, DIM))
    return out


# speedup vs baseline: 1.8891x; 1.0026x over previous
import functools
import math

import jax
import jax.numpy as jnp
from jax import lax
from jax.experimental import pallas as pl
from jax.experimental.pallas import tpu as pltpu
from jax.experimental.pallas import tpu_sc as plsc

DIM = 64
SEQ_LEN = 200
BATCH = 4096
NC = 2
NS = 16
NW = NC * NS
CHUNK = 128
SCALE = math.sqrt(DIM)


def _tc_widen_scale(table):
    n = table.shape[0]
    blk = 16384
    grid = pl.cdiv(n, blk)

    def body(i_ref, o_ref):
        o_ref[:, pl.ds(0, DIM)] = i_ref[...].T * SCALE

    return pl.pallas_call(
        body,
        grid=(grid,),
        in_specs=[pl.BlockSpec((DIM, blk), lambda i: (0, i))],
        out_specs=pl.BlockSpec((blk, 128), lambda i: (i, 0)),
        out_shape=jax.ShapeDtypeStruct((n, 128), jnp.float32),
    )(table.T)


def _make_sc_gather(n_flat):
    per_w = n_flat // NW
    n_chunks = per_w // CHUNK
    assert n_chunks % 4 == 0
    mesh = plsc.VectorSubcoreMesh(core_axis_name="c", subcore_axis_name="s")

    @functools.partial(
        pl.kernel,
        mesh=mesh,
        out_type=jax.ShapeDtypeStruct((n_flat // 512, 32, 8, 128), jnp.float32),
        compiler_params=pltpu.CompilerParams(
            use_tc_tiling_on_sc=False, needs_layout_passes=False),
        scratch_types=[
            pltpu.VMEM((n_chunks, CHUNK), jnp.int32),
            pltpu.VMEM((CHUNK, 128), jnp.float32),
            pltpu.VMEM((CHUNK, 128), jnp.float32),
            pltpu.VMEM((CHUNK, 128), jnp.float32),
            pltpu.VMEM((CHUNK, 128), jnp.float32),
            pltpu.VMEM((8, 8, 133), jnp.float32),
            pltpu.VMEM((8, 8, 133), jnp.float32),
            pltpu.VMEM((DIM,), jnp.float32),
            pltpu.SemaphoreType.DMA,
            pltpu.SemaphoreType.DMA,
            pltpu.SemaphoreType.DMA,
            pltpu.SemaphoreType.DMA,
            pltpu.SemaphoreType.DMA,
            pltpu.SemaphoreType.DMA,
        ],
    )
    def body(idx_hbm, table_hbm, pe_hbm, out_hbm, idx_v,
             slab0, slab1, slab2, slab3, outb0, outb1, peb,
             sg0, sg1, sg2, sg3, sw0, sw1):
        wid = lax.axis_index("c") * NS + lax.axis_index("s")
        w_row = wid * n_chunks
        w_base = wid * per_w
        slabs = (slab0, slab1, slab2, slab3)
        outbs = (outb0, outb1)
        sg = (sg0, sg1, sg2, sg3)
        sw = (sw0, sw1)
        iota = lax.iota(jnp.int32, 16)
        dks = [iota + 16 * k for k in range(4)]
        i0s = [d >> 3 for d in dks]
        i1s = [d & 7 for d in dks]

        pltpu.sync_copy(idx_hbm.at[pl.ds(w_row, n_chunks)], idx_v)

        def gather(c, b):
            pltpu.async_copy(table_hbm.at[idx_v.at[c]], slabs[b], sg[b])

        def wait_g(b):
            pltpu.make_async_copy(
                table_hbm.at[idx_v.at[0]], slabs[b], sg[b]).wait()

        def wait_w(b):
            pltpu.make_async_copy(
                outbs[b].at[:, :, pl.ds(0, 128)],
                out_hbm.at[pl.ds(0, 8), 0], sw[b]).wait()

        s0 = w_base // BATCH
        pltpu.sync_copy(pe_hbm.at[s0], peb)
        gather(0, 0)
        gather(1, 1)
        gather(2, 2)

        def outer(c4, s_prev):
            for b in (0, 1, 2, 3):
                c = c4 * 4 + b
                base = w_base + c * CHUNK
                s = base // BATCH
                cb = (base // CHUNK) % 32

                @pl.when(c + 3 < n_chunks)
                def _():
                    gather(c + 3, (b + 3) % 4)

                @pl.when(s != s_prev)
                def _():
                    pltpu.sync_copy(pe_hbm.at[s], peb)

                wait_g(b)

                @pl.when(c >= 2)
                def _():
                    wait_w(b % 2)

                slab = slabs[b]
                outb = outbs[b % 2]
                pes = [peb[pl.ds(16 * k, 16)] for k in range(4)]

                @plsc.parallel_loop(0, CHUNK, 1, unroll=4)
                def _(r):
                    rv = jnp.broadcast_to(r, (16,)).astype(jnp.int32)
                    for k in range(4):
                        vals = slab[r, pl.ds(16 * k, 16)] + pes[k]
                        plsc.store_scatter(outb, [i0s[k], i1s[k], rv], vals)

                pltpu.async_copy(outb.at[:, :, pl.ds(0, 128)],
                                 out_hbm.at[pl.ds(s * 8, 8), cb], sw[b % 2])
                s_prev = s
            return s_prev

        lax.fori_loop(0, n_chunks // 4, outer, s0)
        wait_w(0)
        wait_w(1)

    return body


def kernel(x, emb_weight, positional_encodings):
    seq, batch = x.shape
    idx2d = x.reshape(-1, CHUNK)
    pe2d = positional_encodings.reshape(positional_encodings.shape[0], DIM)[:seq]
    table_wide = _tc_widen_scale(emb_weight)
    out4 = _make_sc_gather(seq * batch)(idx2d, table_wide, pe2d)
    out = (out4.reshape(seq, 8, 32, 8, 128)
               .transpose(0, 2, 4, 1, 3)
               .reshape(seq, batch, DIM))
    return out
